# baseline (speedup 1.0000x reference)
import functools

import jax
import jax.numpy as jnp
from jax import lax
from jax.experimental import pallas as pl
from jax.experimental.pallas import tpu as pltpu
from jax.experimental.pallas import tpu_sc as plsc


@functools.cache
def _make_sc_broadcast_copy(batch, seq_len, dim, dtype):
    info = plsc.get_sparse_core_info()
    num_cores, num_subcores = info.num_cores, info.num_subcores
    num_workers = num_cores * num_subcores
    assert seq_len % num_workers == 0
    rows_per_worker = seq_len // num_workers
    mesh = plsc.VectorSubcoreMesh(core_axis_name="c", subcore_axis_name="s")

    chunk = min(rows_per_worker, 32)
    nbuf = 3
    assert rows_per_worker % chunk == 0
    num_chunks = rows_per_worker // chunk
    nbuf = min(nbuf, num_chunks)

    @functools.partial(
        pl.kernel,
        mesh=mesh,
        out_type=jax.ShapeDtypeStruct((batch, seq_len, dim), dtype),
        scratch_types=[
            pltpu.VMEM((nbuf, chunk, dim), dtype),
            pltpu.SemaphoreType.DMA,
            pltpu.SemaphoreType.DMA,
        ],
    )
    def sc_copy(w_hbm, out_hbm, bufs, in_sem, out_sem):
        wid = lax.axis_index("s") * num_cores + lax.axis_index("c")
        base = wid * rows_per_worker

        def start_in(i):
            return pltpu.async_copy(
                w_hbm.at[pl.ds(base + i * chunk, chunk)], bufs.at[i % nbuf], in_sem
            )

        def start_outs(i):
            return [
                pltpu.async_copy(
                    bufs.at[i % nbuf],
                    out_hbm.at[b, pl.ds(base + i * chunk, chunk)],
                    out_sem,
                )
                for b in range(batch)
            ]

        in_h = [start_in(i) for i in range(nbuf)]
        outs = [None] * num_chunks
        for i in range(num_chunks):
            in_h[i % nbuf].wait()
            j = i - (nbuf - 1)
            if j >= 0 and outs[j]:
                for h in outs[j]:
                    h.wait()
                outs[j] = None
            if nbuf <= i + 1 < num_chunks:
                in_h[(i + 1) % nbuf] = start_in(i + 1)
            outs[i] = start_outs(i)
        for pend in outs:
            if pend:
                for h in pend:
                    h.wait()

    return sc_copy


def kernel(input_seq, weights):
    batch, seq_len = input_seq.shape
    dim = weights.shape[1]
    fn = _make_sc_broadcast_copy(batch, seq_len, dim, weights.dtype)
    return fn(weights)


# speedup vs baseline: 1.0521x; 1.0521x over previous
import functools

import jax
import jax.numpy as jnp
from jax import lax
from jax.experimental import pallas as pl
from jax.experimental.pallas import tpu as pltpu
from jax.experimental.pallas import tpu_sc as plsc


@functools.cache
def _make_sc_broadcast_copy(batch, seq_len, dim, dtype):
    info = plsc.get_sparse_core_info()
    num_cores, num_subcores = info.num_cores, info.num_subcores
    num_workers = num_cores * num_subcores
    assert seq_len % num_workers == 0
    rows_per_worker = seq_len // num_workers
    mesh = plsc.VectorSubcoreMesh(core_axis_name="c", subcore_axis_name="s")

    chunk = min(rows_per_worker, 64)
    nbuf = 2
    assert rows_per_worker % chunk == 0
    num_chunks = rows_per_worker // chunk
    nbuf = min(nbuf, num_chunks)

    @functools.partial(
        pl.kernel,
        mesh=mesh,
        out_type=jax.ShapeDtypeStruct((batch, seq_len, dim), dtype),
        scratch_types=[
            pltpu.VMEM((nbuf, chunk, dim), dtype),
            pltpu.SemaphoreType.DMA,
            pltpu.SemaphoreType.DMA,
        ],
    )
    def sc_copy(w_hbm, out_hbm, bufs, in_sem, out_sem):
        wid = lax.axis_index("s") * num_cores + lax.axis_index("c")
        base = wid * rows_per_worker

        def start_in(i):
            return pltpu.async_copy(
                w_hbm.at[pl.ds(base + i * chunk, chunk)], bufs.at[i % nbuf], in_sem
            )

        def start_outs(i):
            return [
                pltpu.async_copy(
                    bufs.at[i % nbuf],
                    out_hbm.at[b, pl.ds(base + i * chunk, chunk)],
                    out_sem,
                )
                for b in range(batch)
            ]

        in_h = [start_in(i) for i in range(nbuf)]
        outs = [None] * num_chunks
        for i in range(num_chunks):
            in_h[i % nbuf].wait()
            outs[i] = start_outs(i)
            j = i + 1 - nbuf
            if j >= 0 and outs[j]:
                for h in outs[j]:
                    h.wait()
                outs[j] = None
            if nbuf <= i + 1 < num_chunks:
                in_h[(i + 1) % nbuf] = start_in(i + 1)
        for pend in outs:
            if pend:
                for h in pend:
                    h.wait()

    return sc_copy


def kernel(input_seq, weights):
    batch, seq_len = input_seq.shape
    dim = weights.shape[1]
    fn = _make_sc_broadcast_copy(batch, seq_len, dim, weights.dtype)
    return fn(weights)
